# Initial kernel scaffold; baseline (speedup 1.0000x reference)
import jax
import jax.numpy as jnp
from jax.experimental import pallas as pl


def kernel(state, weight_matrix):
    raise NotImplementedError("write your pallas kernel here")


# speedup vs baseline: 1.4388x; 1.4388x over previous
import functools

import jax
import jax.numpy as jnp
from jax import lax
from jax.experimental import pallas as pl
from jax.experimental.pallas import tpu as pltpu

T_STEPS = 16


def _body(w_ref, state_ref, out_ref, acc_ref):
    ns = pl.program_id(0)

    @pl.when(ns == 0)
    def _():
        acc_ref[...] = jnp.zeros_like(acc_ref)

    w = w_ref[0, :].astype(jnp.bfloat16).astype(jnp.float32)
    blk = state_ref[...].astype(jnp.bfloat16).astype(jnp.float32)
    acc_ref[...] += jnp.sum(blk * w[None, :, None], axis=1)

    @pl.when(ns == pl.num_programs(0) - 1)
    def _():
        s = acc_ref[...]
        b, d = s.shape
        m = jnp.max(s, axis=-1, keepdims=True)
        e = jnp.exp(s - m)
        p = e / jnp.sum(e, axis=-1, keepdims=True)
        iota = lax.broadcasted_iota(jnp.int32, (b, d), 1)
        cur = p
        out_ref[:, 0, :] = cur
        for t in range(1, T_STEPS):
            mx = jnp.max(cur, axis=-1, keepdims=True)
            first = jnp.min(
                jnp.where(cur == mx, iota, d), axis=-1, keepdims=True
            )
            masked = jnp.where(iota == first, 0.0, cur)
            cur = masked / jnp.sum(masked, axis=-1, keepdims=True)
            out_ref[:, t, :] = cur


@functools.partial(jax.jit, static_argnames=("interpret",))
def kernel(state, weight_matrix, interpret=False):
    B, S, D = state.shape
    chunk = 128
    ns = S // chunk
    out = pl.pallas_call(
        _body,
        grid=(ns,),
        in_specs=[
            pl.BlockSpec((1, chunk), lambda i: (0, i)),
            pl.BlockSpec((B, chunk, D), lambda i: (0, i, 0)),
        ],
        out_specs=pl.BlockSpec((B, T_STEPS, D), lambda i: (0, 0, 0)),
        out_shape=jax.ShapeDtypeStruct((B, T_STEPS, D), jnp.float32),
        scratch_shapes=[pltpu.VMEM((B, D), jnp.float32)],
        compiler_params=pltpu.CompilerParams(
            dimension_semantics=("arbitrary",),
        ),
        interpret=interpret,
    )(weight_matrix, state)
    return out
